# Initial kernel scaffold; baseline (speedup 1.0000x reference)
import jax
import jax.numpy as jnp
from jax.experimental import pallas as pl


def kernel(images, boxes, patch):
    raise NotImplementedError("write your pallas kernel here")


# speedup vs baseline: 171.7253x; 171.7253x over previous
import jax
import jax.numpy as jnp
import numpy as np
from jax.experimental import pallas as pl
from jax.experimental.pallas import tpu as pltpu

_ASPECT = 1.0
_SCALE = 0.3
_MIN_PATCH_H = 60.0
_EPS_TOTAL = 1000.0 * float(np.finfo(np.float32).eps)
_SPAN = 128


def _weight_mat(in_size, out_len, out_size_f, shift):
    inv_scale = in_size / out_size_f
    kernel_scale = jnp.maximum(inv_scale, 1.0)
    ocoord = jax.lax.broadcasted_iota(jnp.int32, (in_size, out_len), 1)
    ocoord = (ocoord - shift).astype(jnp.float32)
    sample_f = (ocoord + 0.5) * inv_scale - 0.5
    a = jax.lax.broadcasted_iota(jnp.int32, (in_size, out_len), 0).astype(
        jnp.float32)
    x = jnp.abs(sample_f - a) / kernel_scale
    w = jnp.maximum(0.0, 1.0 - x)
    total = jnp.sum(w, axis=0, keepdims=True)
    w = jnp.where(jnp.abs(total) > _EPS_TOTAL,
                  w / jnp.where(total != 0, total, 1.0), 0.0)
    valid = (sample_f >= -0.5) & (sample_f <= in_size - 0.5)
    return jnp.where(valid, w, 0.0)


def _patcher_body(boxes_ref, img_ref, patch_ref, out_ref):
    C, H, W = out_ref.shape
    PH, PW = patch_ref.shape[1], patch_ref.shape[2]
    NB = boxes_ref.shape[0]

    out_ref[...] = img_ref[...]

    p = patch_ref[...]
    mp = jnp.mean(p, axis=(1, 2), keepdims=True)
    sp = jnp.sqrt(jnp.mean((p - mp) ** 2, axis=(1, 2), keepdims=True)) + 1e-6
    pn = (p - mp) / sp

    jy = jax.lax.broadcasted_iota(jnp.int32, (_SPAN, W), 0)
    kx = jax.lax.broadcasted_iota(jnp.int32, (_SPAN, W), 1)

    hi = jax.lax.Precision.HIGHEST
    dn = (((0,), (0,)), ((), ()))

    def box_step(n, carry):
        ymin = boxes_ref[n, 0]
        xmin = boxes_ref[n, 1]
        ymax = boxes_ref[n, 2]
        xmax = boxes_ref[n, 3]
        h = ymax - ymin
        w = xmax - xmin
        pwf = h * _SCALE
        phf = _ASPECT * pwf
        oy = ymin + h / 2.0
        ox = xmin + w / 2.0
        yp = jnp.maximum(oy - phf / 2.0, 0.0)
        xp = jnp.maximum(ox - pwf / 2.0, 0.0)
        yp = jnp.where(yp + phf > H, H - phf, yp)
        xp = jnp.where(xp + pwf > W, W - pwf, xp)
        yp_i = yp.astype(jnp.int32)
        xp_i = xp.astype(jnp.int32)
        ph_i = phf.astype(jnp.int32)
        pw_i = pwf.astype(jnp.int32)

        a_y = jnp.minimum((yp_i // 8) * 8, H - _SPAN)
        dy = yp_i - a_y

        slab = out_ref[:, pl.ds(a_y, _SPAN), :]

        rmask = (jy >= dy) & (jy < dy + ph_i)
        cmask = (kx >= xp_i) & (kx < xp_i + pw_i)
        mask = rmask & cmask
        cnt = (ph_i * pw_i).astype(jnp.float32)
        s1 = jnp.sum(jnp.where(mask[None], slab, 0.0), axis=(1, 2),
                     keepdims=True)
        mb = s1 / cnt
        s2 = jnp.sum(jnp.where(mask[None], (slab - mb) ** 2, 0.0),
                     axis=(1, 2), keepdims=True)
        sb = jnp.sqrt(s2 / cnt)

        m = pn * sb + mb

        wy = _weight_mat(PH, _SPAN, phf.astype(jnp.int32).astype(jnp.float32),
                         dy)
        wx = _weight_mat(PW, W, pwf.astype(jnp.int32).astype(jnp.float32),
                         xp_i)

        ims = []
        for c in range(C):
            t = jax.lax.dot_general(m[c], wy, dn, precision=hi,
                                    preferred_element_type=jnp.float32)
            im = jax.lax.dot_general(t, wx, dn, precision=hi,
                                     preferred_element_type=jnp.float32)
            ims.append(im)
        im3 = jnp.stack(ims, axis=0)

        wmask = mask & (phf > _MIN_PATCH_H)
        out_ref[:, pl.ds(a_y, _SPAN), :] = jnp.where(wmask[None], im3, slab)
        return carry

    jax.lax.fori_loop(0, NB, box_step, 0)


def kernel(images, boxes, patch):
    B, H, W, C = images.shape
    NB = boxes.shape[1]
    PH, PW = patch.shape[0], patch.shape[1]
    imgs = jnp.transpose(images, (0, 3, 1, 2))
    pat = jnp.transpose(patch, (2, 0, 1))
    out = pl.pallas_call(
        _patcher_body,
        grid=(B,),
        in_specs=[
            pl.BlockSpec((None, NB, 4), lambda b: (b, 0, 0),
                         memory_space=pltpu.SMEM),
            pl.BlockSpec((None, C, H, W), lambda b: (b, 0, 0, 0)),
            pl.BlockSpec((C, PH, PW), lambda b: (0, 0, 0)),
        ],
        out_specs=pl.BlockSpec((None, C, H, W), lambda b: (b, 0, 0, 0)),
        out_shape=jax.ShapeDtypeStruct((B, C, H, W), images.dtype),
    )(boxes, imgs, pat)
    return jnp.transpose(out, (0, 2, 3, 1))


# speedup vs baseline: 301.0637x; 1.7532x over previous
import jax
import jax.numpy as jnp
import numpy as np
from jax.experimental import pallas as pl
from jax.experimental.pallas import tpu as pltpu

_ASPECT = 1.0
_SCALE = 0.3
_MIN_PATCH_H = 60.0
_EPS_TOTAL = 1000.0 * float(np.finfo(np.float32).eps)
_SPAN = 128


def _weight_mat(in_size, out_len, out_size_f, shift):
    inv_scale = in_size / out_size_f
    kernel_scale = jnp.maximum(inv_scale, 1.0)
    ocoord = jax.lax.broadcasted_iota(jnp.int32, (in_size, out_len), 1)
    ocoord = (ocoord - shift).astype(jnp.float32)
    sample_f = (ocoord + 0.5) * inv_scale - 0.5
    a = jax.lax.broadcasted_iota(jnp.int32, (in_size, out_len), 0).astype(
        jnp.float32)
    x = jnp.abs(sample_f - a) / kernel_scale
    w = jnp.maximum(0.0, 1.0 - x)
    total = jnp.sum(w, axis=0, keepdims=True)
    w = jnp.where(jnp.abs(total) > _EPS_TOTAL,
                  w / jnp.where(total != 0, total, 1.0), 0.0)
    valid = (sample_f >= -0.5) & (sample_f <= in_size - 0.5)
    return jnp.where(valid, w, 0.0)


def _patcher_body(boxes_ref, img_ref, patch_ref, out_ref):
    C, H, W = out_ref.shape
    PH, PW = patch_ref.shape[1], patch_ref.shape[2]
    NB = boxes_ref.shape[0]

    out_ref[...] = img_ref[...]

    p = patch_ref[...]
    mp = jnp.mean(p, axis=(1, 2), keepdims=True)
    sp = jnp.sqrt(jnp.mean((p - mp) ** 2, axis=(1, 2), keepdims=True)) + 1e-6
    pn = (p - mp) / sp

    CSPAN = 256
    jy = jax.lax.broadcasted_iota(jnp.int32, (_SPAN, CSPAN), 0)
    kx = jax.lax.broadcasted_iota(jnp.int32, (_SPAN, CSPAN), 1)

    hi = jax.lax.Precision.HIGHEST
    dn = (((0,), (0,)), ((), ()))

    def box_step(n, carry):
        ymin = boxes_ref[n, 0]
        xmin = boxes_ref[n, 1]
        ymax = boxes_ref[n, 2]
        xmax = boxes_ref[n, 3]
        h = ymax - ymin
        w = xmax - xmin
        pwf = h * _SCALE
        phf = _ASPECT * pwf
        oy = ymin + h / 2.0
        ox = xmin + w / 2.0
        yp = jnp.maximum(oy - phf / 2.0, 0.0)
        xp = jnp.maximum(ox - pwf / 2.0, 0.0)
        yp = jnp.where(yp + phf > H, H - phf, yp)
        xp = jnp.where(xp + pwf > W, W - pwf, xp)
        yp_i = yp.astype(jnp.int32)
        xp_i = xp.astype(jnp.int32)
        ph_i = phf.astype(jnp.int32)
        pw_i = pwf.astype(jnp.int32)

        a_y = jnp.minimum((yp_i // 8) * 8, H - _SPAN)
        dy = yp_i - a_y
        a_x = jnp.minimum((xp_i // 128) * 128, W - CSPAN)
        dx = xp_i - a_x

        slab = out_ref[:, pl.ds(a_y, _SPAN), pl.ds(a_x, CSPAN)]

        rmask = (jy >= dy) & (jy < dy + ph_i)
        cmask = (kx >= dx) & (kx < dx + pw_i)
        mask = rmask & cmask
        cnt = (ph_i * pw_i).astype(jnp.float32)
        s1 = jnp.sum(jnp.where(mask[None], slab, 0.0), axis=(1, 2),
                     keepdims=True)
        mb = s1 / cnt
        s2 = jnp.sum(jnp.where(mask[None], (slab - mb) ** 2, 0.0),
                     axis=(1, 2), keepdims=True)
        sb = jnp.sqrt(s2 / cnt)

        m = pn * sb + mb

        wy = _weight_mat(PH, _SPAN, phf.astype(jnp.int32).astype(jnp.float32),
                         dy)
        wx = _weight_mat(PW, CSPAN, pwf.astype(jnp.int32).astype(jnp.float32),
                         dx)

        ims = []
        for c in range(C):
            t = jax.lax.dot_general(m[c], wy, dn, precision=hi,
                                    preferred_element_type=jnp.float32)
            im = jax.lax.dot_general(t, wx, dn, precision=hi,
                                     preferred_element_type=jnp.float32)
            ims.append(im)
        im3 = jnp.stack(ims, axis=0)

        wmask = mask & (phf > _MIN_PATCH_H)
        out_ref[:, pl.ds(a_y, _SPAN), pl.ds(a_x, CSPAN)] = jnp.where(
            wmask[None], im3, slab)
        return carry

    jax.lax.fori_loop(0, NB, box_step, 0)


def kernel(images, boxes, patch):
    B, H, W, C = images.shape
    NB = boxes.shape[1]
    PH, PW = patch.shape[0], patch.shape[1]
    imgs = jnp.transpose(images, (0, 3, 1, 2))
    pat = jnp.transpose(patch, (2, 0, 1))
    out = pl.pallas_call(
        _patcher_body,
        grid=(B,),
        in_specs=[
            pl.BlockSpec((None, NB, 4), lambda b: (b, 0, 0),
                         memory_space=pltpu.SMEM),
            pl.BlockSpec((None, C, H, W), lambda b: (b, 0, 0, 0)),
            pl.BlockSpec((C, PH, PW), lambda b: (0, 0, 0)),
        ],
        out_specs=pl.BlockSpec((None, C, H, W), lambda b: (b, 0, 0, 0)),
        out_shape=jax.ShapeDtypeStruct((B, C, H, W), images.dtype),
    )(boxes, imgs, pat)
    return jnp.transpose(out, (0, 2, 3, 1))
